# baseline (speedup 1.0000x reference)
import functools

import jax
import jax.numpy as jnp
from jax import lax
from jax.experimental import pallas as pl
from jax.experimental.pallas import tpu as pltpu
from jax.experimental.pallas import tpu_sc as plsc

_R16 = 16
_D = 32
_L = 5
_B = 1024
_PB = 2048


def _gru_kernel(pathT_ref, all16_ref, edge_ref, W_ih_ref, W_hh_ref,
                b_ih_ref, b_hh_ref, W_lin_ref, b_lin_ref, out_ref):
    f32 = jnp.float32
    dot = functools.partial(jax.lax.dot_general,
                            preferred_element_type=jnp.float32)
    mm_nt = (((1,), (1,)), ((), ()))
    W_ih = W_ih_ref[...]
    Ga = dot(W_ih, all16_ref[...], mm_nt)
    Ge = dot(W_ih, edge_ref[...], mm_nt)
    W_hh = W_hh_ref[...]
    b_ih = b_ih_ref[...]
    b_hh = b_hh_ref[...]

    mm = (((1,), (0,)), ((), ()))
    hT = jnp.zeros((_D, _PB), dtype=f32)
    iota16 = jax.lax.broadcasted_iota(jnp.int32, (_R16, _PB), 0)
    for t in range(_L):
        ids = pathT_ref[t, :]
        oh = (iota16 == ids[None, :]).astype(f32)
        G = Ga if t % 2 == 0 else Ge
        giT = dot(G, oh, mm) + b_ih
        ghT = dot(W_hh, hT, mm) + b_hh
        r = 0.5 * jnp.tanh(0.5 * (giT[0:_D] + ghT[0:_D])) + 0.5
        z = 0.5 * jnp.tanh(0.5 * (giT[_D:2 * _D] + ghT[_D:2 * _D])) + 0.5
        n = jnp.tanh(giT[2 * _D:] + r * ghT[2 * _D:])
        hT = (1.0 - z) * n + z * hT

    out_ref[...] = dot(W_lin_ref[...], hT, mm) + b_lin_ref[...]


def _tc_per_path(path, all16, edge_embedding, W_ih, W_hh, b_ih, b_hh,
                 W_lin, b_lin):
    P = path.shape[0]
    pathT = path.T
    const = lambda *_: (0, 0)
    outT = pl.pallas_call(
        _gru_kernel,
        grid=(P // _PB,),
        in_specs=[
            pl.BlockSpec((_L, _PB), lambda i: (0, i)),
            pl.BlockSpec((_R16, _D), const),
            pl.BlockSpec((_R16, _D), const),
            pl.BlockSpec((3 * _D, _D), const),
            pl.BlockSpec((3 * _D, _D), const),
            pl.BlockSpec((3 * _D, 1), const),
            pl.BlockSpec((3 * _D, 1), const),
            pl.BlockSpec((1, _D), const),
            pl.BlockSpec((1, 1), const),
        ],
        out_specs=pl.BlockSpec((1, _PB), lambda i: (0, i)),
        out_shape=jax.ShapeDtypeStruct((1, P), jnp.float32),
    )(pathT, all16, edge_embedding, W_ih, W_hh,
      b_ih.reshape(3 * _D, 1), b_hh.reshape(3 * _D, 1),
      W_lin, b_lin.reshape(1, 1))
    return outT.reshape(P)


def _make_sc_segsum(P, interpret=False):
    NW = 16
    chunk = P // NW
    nvec = chunk // 16
    cols = 128
    nred = _B // cols
    mesh = plsc.VectorSubcoreMesh(core_axis_name="c", subcore_axis_name="s",
                                  num_cores=1)

    @functools.partial(
        pl.kernel, mesh=mesh,
        out_type=jax.ShapeDtypeStruct((_B,), jnp.float32),
        compiler_params=pltpu.CompilerParams(needs_layout_passes=False),
        interpret=interpret,
        scratch_types=[
            pltpu.VMEM((chunk,), jnp.float32),
            pltpu.VMEM((chunk + 32,), jnp.int32),
            pltpu.VMEM((_B,), jnp.float32),
            pltpu.VMEM((NW, cols), jnp.float32),
            pltpu.VMEM((32,), jnp.float32),
            pltpu.VMEM_SHARED((NW, _B), jnp.float32),
        ],
    )
    def seg(vals_hbm, idx_hbm, out_hbm, vals_v, idx_v, acc_v, red_v, pad_v,
            shared):
        w = lax.axis_index("s")
        base = w * chunk
        f32 = jnp.float32
        zeros16 = jnp.zeros((16,), f32)
        pad_v[pl.ds(0, 16)] = zeros16
        idx_v[pl.ds(0, 16)] = jnp.full((16,), -1, jnp.int32)
        idx_v[pl.ds(chunk + 16, 16)] = jnp.full((16,), -2, jnp.int32)
        pltpu.sync_copy(vals_hbm.at[pl.ds(base, chunk)], vals_v)
        pltpu.sync_copy(idx_hbm.at[pl.ds(base, chunk)],
                        idx_v.at[pl.ds(16, chunk)])

        def zero_body(i, _):
            acc_v[pl.ds(i * 16, 16)] = zeros16
            return 0
        lax.fori_loop(0, _B // 16, zero_body, 0)

        def body(i, _):
            v = vals_v[pl.ds(i * 16, 16)]
            ix = idx_v[pl.ds(16 + i * 16, 16)]
            ixm1 = idx_v[pl.ds(15 + i * 16, 16)]
            ixp1 = idx_v[pl.ds(17 + i * 16, 16)]
            p = v
            for shift in (1, 2, 4, 8):
                pad_v[pl.ds(16, 16)] = p
                p = p + pad_v[pl.ds(16 - shift, 16)]
            e = p - v
            lane = lax.iota(jnp.int32, 16)
            first = (ix != ixm1) | (lane == 0)
            last = (ix != ixp1) | (lane == 15)
            plsc.addupdate_scatter(acc_v, [ix], p, mask=last)
            plsc.addupdate_scatter(acc_v, [ix], -e, mask=first)
            return 0
        lax.fori_loop(0, nvec, body, 0)

        pltpu.sync_copy(acc_v, shared.at[w])
        plsc.subcore_barrier()

        @pl.when(w < nred)
        def _reduce():
            pltpu.sync_copy(shared.at[:, pl.ds(w * cols, cols)], red_v)

            def rz(j, _):
                acc_v[pl.ds(j * 16, 16)] = zeros16
                return 0
            lax.fori_loop(0, cols // 16, rz, 0)

            def rbody(r, _):
                def rj(j, _2):
                    acc_v[pl.ds(j * 16, 16)] = (acc_v[pl.ds(j * 16, 16)] +
                                                red_v[r, pl.ds(j * 16, 16)])
                    return 0
                return lax.fori_loop(0, cols // 16, rj, 0)
            lax.fori_loop(0, NW, rbody, 0)

            pltpu.sync_copy(acc_v.at[pl.ds(0, cols)],
                            out_hbm.at[pl.ds(w * cols, cols)])

    return seg


def kernel(users, path, path_idx, all_embedding, edge_embedding,
           virtual_embedding, W_ih, W_hh, b_ih, b_hh, W_lin, b_lin):
    del users, virtual_embedding
    P = path.shape[0]
    all16 = jax.lax.slice(all_embedding, (0, 0), (_R16, _D))
    out = _tc_per_path(path, all16, edge_embedding, W_ih, W_hh, b_ih, b_hh,
                       W_lin, b_lin)
    score = _make_sc_segsum(P)(out, path_idx)
    return score.reshape(_B, 1)


# speedup vs baseline: 1.0120x; 1.0120x over previous
import functools

import jax
import jax.numpy as jnp
from jax import lax
from jax.experimental import pallas as pl
from jax.experimental.pallas import tpu as pltpu
from jax.experimental.pallas import tpu_sc as plsc

_R16 = 16
_D = 32
_L = 5
_B = 1024
_PB = 2048


def _gru_kernel(pathT_ref, all16_ref, edge_ref, W_ih_ref, W_hh_ref,
                b_ih_ref, b_hh_ref, W_lin_ref, b_lin_ref, out_ref):
    f32 = jnp.float32
    dot = functools.partial(jax.lax.dot_general,
                            preferred_element_type=jnp.float32)
    mm_nt = (((1,), (1,)), ((), ()))
    W_ih = W_ih_ref[...]
    Ga = dot(W_ih, all16_ref[...], mm_nt)
    Ge = dot(W_ih, edge_ref[...], mm_nt)
    W_hh = W_hh_ref[...]
    b_ih = b_ih_ref[...]
    b_hh = b_hh_ref[...]

    mm = (((1,), (0,)), ((), ()))
    hT = jnp.zeros((_D, _PB), dtype=f32)
    iota16 = jax.lax.broadcasted_iota(jnp.int32, (_R16, _PB), 0)
    for t in range(_L):
        ids = pathT_ref[t, :]
        oh = (iota16 == ids[None, :]).astype(f32)
        G = Ga if t % 2 == 0 else Ge
        giT = dot(G, oh, mm) + b_ih
        ghT = dot(W_hh, hT, mm) + b_hh
        r = jax.nn.sigmoid(giT[0:_D] + ghT[0:_D])
        z = jax.nn.sigmoid(giT[_D:2 * _D] + ghT[_D:2 * _D])
        n = jnp.tanh(giT[2 * _D:] + r * ghT[2 * _D:])
        hT = (1.0 - z) * n + z * hT

    out_ref[...] = dot(W_lin_ref[...], hT, mm) + b_lin_ref[...]


def _tc_per_path(path, all16, edge_embedding, W_ih, W_hh, b_ih, b_hh,
                 W_lin, b_lin):
    P = path.shape[0]
    pathT = path.T
    const = lambda *_: (0, 0)
    outT = pl.pallas_call(
        _gru_kernel,
        grid=(P // _PB,),
        in_specs=[
            pl.BlockSpec((_L, _PB), lambda i: (0, i)),
            pl.BlockSpec((_R16, _D), const),
            pl.BlockSpec((_R16, _D), const),
            pl.BlockSpec((3 * _D, _D), const),
            pl.BlockSpec((3 * _D, _D), const),
            pl.BlockSpec((3 * _D, 1), const),
            pl.BlockSpec((3 * _D, 1), const),
            pl.BlockSpec((1, _D), const),
            pl.BlockSpec((1, 1), const),
        ],
        out_specs=pl.BlockSpec((1, _PB), lambda i: (0, i)),
        out_shape=jax.ShapeDtypeStruct((1, P), jnp.float32),
    )(pathT, all16, edge_embedding, W_ih, W_hh,
      b_ih.reshape(3 * _D, 1), b_hh.reshape(3 * _D, 1),
      W_lin, b_lin.reshape(1, 1))
    return outT.reshape(P)


def _make_sc_segsum(P, interpret=False):
    NW = 16
    chunk = P // NW
    nvec = chunk // 16
    cols = 128
    nred = _B // cols
    mesh = plsc.VectorSubcoreMesh(core_axis_name="c", subcore_axis_name="s",
                                  num_cores=1)

    @functools.partial(
        pl.kernel, mesh=mesh,
        out_type=jax.ShapeDtypeStruct((_B,), jnp.float32),
        compiler_params=pltpu.CompilerParams(needs_layout_passes=False),
        interpret=interpret,
        scratch_types=[
            pltpu.VMEM((chunk,), jnp.float32),
            pltpu.VMEM((chunk + 32,), jnp.int32),
            pltpu.VMEM((_B,), jnp.float32),
            pltpu.VMEM((NW, cols), jnp.float32),
            pltpu.VMEM((32,), jnp.float32),
            pltpu.VMEM_SHARED((NW, _B), jnp.float32),
        ],
    )
    def seg(vals_hbm, idx_hbm, out_hbm, vals_v, idx_v, acc_v, red_v, pad_v,
            shared):
        w = lax.axis_index("s")
        base = w * chunk
        f32 = jnp.float32
        zeros16 = jnp.zeros((16,), f32)
        pad_v[pl.ds(0, 16)] = zeros16
        idx_v[pl.ds(0, 16)] = jnp.full((16,), -1, jnp.int32)
        idx_v[pl.ds(chunk + 16, 16)] = jnp.full((16,), -2, jnp.int32)
        pltpu.sync_copy(vals_hbm.at[pl.ds(base, chunk)], vals_v)
        pltpu.sync_copy(idx_hbm.at[pl.ds(base, chunk)],
                        idx_v.at[pl.ds(16, chunk)])

        def zero_body(i, _):
            acc_v[pl.ds(i * 16, 16)] = zeros16
            return 0
        lax.fori_loop(0, _B // 16, zero_body, 0)

        def body(i, _):
            v = vals_v[pl.ds(i * 16, 16)]
            ix = idx_v[pl.ds(16 + i * 16, 16)]
            ixm1 = idx_v[pl.ds(15 + i * 16, 16)]
            ixp1 = idx_v[pl.ds(17 + i * 16, 16)]
            p = v
            for shift in (1, 2, 4, 8):
                pad_v[pl.ds(16, 16)] = p
                p = p + pad_v[pl.ds(16 - shift, 16)]
            e = p - v
            lane = lax.iota(jnp.int32, 16)
            first = (ix != ixm1) | (lane == 0)
            last = (ix != ixp1) | (lane == 15)
            plsc.addupdate_scatter(acc_v, [ix], p, mask=last)
            plsc.addupdate_scatter(acc_v, [ix], -e, mask=first)
            return 0
        lax.fori_loop(0, nvec, body, 0, unroll=4)

        pltpu.sync_copy(acc_v, shared.at[w])
        plsc.subcore_barrier()

        @pl.when(w < nred)
        def _reduce():
            pltpu.sync_copy(shared.at[:, pl.ds(w * cols, cols)], red_v)

            def rz(j, _):
                acc_v[pl.ds(j * 16, 16)] = zeros16
                return 0
            lax.fori_loop(0, cols // 16, rz, 0)

            def rbody(r, _):
                def rj(j, _2):
                    acc_v[pl.ds(j * 16, 16)] = (acc_v[pl.ds(j * 16, 16)] +
                                                red_v[r, pl.ds(j * 16, 16)])
                    return 0
                return lax.fori_loop(0, cols // 16, rj, 0)
            lax.fori_loop(0, NW, rbody, 0)

            pltpu.sync_copy(acc_v.at[pl.ds(0, cols)],
                            out_hbm.at[pl.ds(w * cols, cols)])

    return seg


def kernel(users, path, path_idx, all_embedding, edge_embedding,
           virtual_embedding, W_ih, W_hh, b_ih, b_hh, W_lin, b_lin):
    del users, virtual_embedding
    P = path.shape[0]
    all16 = jax.lax.slice(all_embedding, (0, 0), (_R16, _D))
    out = _tc_per_path(path, all16, edge_embedding, W_ih, W_hh, b_ih, b_hh,
                       W_lin, b_lin)
    score = _make_sc_segsum(P)(out, path_idx)
    return score.reshape(_B, 1)


# speedup vs baseline: 1.0141x; 1.0020x over previous
import functools

import jax
import jax.numpy as jnp
from jax import lax
from jax.experimental import pallas as pl
from jax.experimental.pallas import tpu as pltpu
from jax.experimental.pallas import tpu_sc as plsc

_R16 = 16
_D = 32
_L = 5
_B = 1024
_PB = 2048


def _gru_kernel(pathT_ref, all16_ref, edge_ref, W_ih_ref, W_hh_ref,
                b_ih_ref, b_hh_ref, W_lin_ref, b_lin_ref, out_ref):
    f32 = jnp.float32
    dot = functools.partial(jax.lax.dot_general,
                            preferred_element_type=jnp.float32)
    mm_nt = (((1,), (1,)), ((), ()))
    W_ih = W_ih_ref[...]
    Ga = dot(W_ih, all16_ref[...], mm_nt)
    Ge = dot(W_ih, edge_ref[...], mm_nt)
    W_hh = W_hh_ref[...]
    b_ih = b_ih_ref[...]
    b_hh = b_hh_ref[...]

    mm = (((1,), (0,)), ((), ()))
    hT = jnp.zeros((_D, _PB), dtype=f32)
    iota16 = jax.lax.broadcasted_iota(jnp.int32, (_R16, _PB), 0)
    for t in range(_L):
        ids = pathT_ref[t, :]
        oh = (iota16 == ids[None, :]).astype(f32)
        G = Ga if t % 2 == 0 else Ge
        giT = dot(G, oh, mm) + b_ih
        ghT = dot(W_hh, hT, mm) + b_hh
        r = jax.nn.sigmoid(giT[0:_D] + ghT[0:_D])
        z = jax.nn.sigmoid(giT[_D:2 * _D] + ghT[_D:2 * _D])
        n = jnp.tanh(giT[2 * _D:] + r * ghT[2 * _D:])
        hT = (1.0 - z) * n + z * hT

    out_ref[...] = dot(W_lin_ref[...], hT, mm) + b_lin_ref[...]


def _tc_per_path(path, all16, edge_embedding, W_ih, W_hh, b_ih, b_hh,
                 W_lin, b_lin):
    P = path.shape[0]
    pathT = path.T
    const = lambda *_: (0, 0)
    outT = pl.pallas_call(
        _gru_kernel,
        grid=(P // _PB,),
        in_specs=[
            pl.BlockSpec((_L, _PB), lambda i: (0, i)),
            pl.BlockSpec((_R16, _D), const),
            pl.BlockSpec((_R16, _D), const),
            pl.BlockSpec((3 * _D, _D), const),
            pl.BlockSpec((3 * _D, _D), const),
            pl.BlockSpec((3 * _D, 1), const),
            pl.BlockSpec((3 * _D, 1), const),
            pl.BlockSpec((1, _D), const),
            pl.BlockSpec((1, 1), const),
        ],
        out_specs=pl.BlockSpec((1, _PB), lambda i: (0, i)),
        out_shape=jax.ShapeDtypeStruct((1, P), jnp.float32),
    )(pathT, all16, edge_embedding, W_ih, W_hh,
      b_ih.reshape(3 * _D, 1), b_hh.reshape(3 * _D, 1),
      W_lin, b_lin.reshape(1, 1))
    return outT.reshape(P)


def _make_sc_segsum(P):
    NW = 16
    chunk = P // NW
    nvec = chunk // 16
    cols = 128
    nred = _B // cols
    mesh = plsc.VectorSubcoreMesh(core_axis_name="c", subcore_axis_name="s",
                                  num_cores=1)

    @functools.partial(
        pl.kernel, mesh=mesh,
        out_type=jax.ShapeDtypeStruct((_B,), jnp.float32),
        compiler_params=pltpu.CompilerParams(needs_layout_passes=False),
        scratch_types=[
            pltpu.VMEM((chunk,), jnp.float32),
            pltpu.VMEM((chunk + 32,), jnp.int32),
            pltpu.VMEM((_B,), jnp.float32),
            pltpu.VMEM((NW, cols), jnp.float32),
            pltpu.VMEM((32,), jnp.float32),
            pltpu.VMEM_SHARED((NW, _B), jnp.float32),
        ],
    )
    def seg(vals_hbm, idx_hbm, out_hbm, vals_v, idx_v, acc_v, red_v, pad_v,
            shared):
        w = lax.axis_index("s")
        base = w * chunk
        f32 = jnp.float32
        zeros16 = jnp.zeros((16,), f32)
        pad_v[pl.ds(0, 16)] = zeros16
        idx_v[pl.ds(0, 16)] = jnp.full((16,), -1, jnp.int32)
        idx_v[pl.ds(chunk + 16, 16)] = jnp.full((16,), -2, jnp.int32)
        pltpu.sync_copy(vals_hbm.at[pl.ds(base, chunk)], vals_v)
        pltpu.sync_copy(idx_hbm.at[pl.ds(base, chunk)],
                        idx_v.at[pl.ds(16, chunk)])

        def zero_body(i, _):
            acc_v[pl.ds(i * 16, 16)] = zeros16
            return 0
        lax.fori_loop(0, _B // 16, zero_body, 0)

        def body(i, _):
            v = vals_v[pl.ds(i * 16, 16)]
            ix = idx_v[pl.ds(16 + i * 16, 16)]
            ixm1 = idx_v[pl.ds(15 + i * 16, 16)]
            ixp1 = idx_v[pl.ds(17 + i * 16, 16)]
            p = v
            for shift in (1, 2, 4, 8):
                pad_v[pl.ds(16, 16)] = p
                p = p + pad_v[pl.ds(16 - shift, 16)]
            e = p - v
            lane = lax.iota(jnp.int32, 16)
            first = (ix != ixm1) | (lane == 0)
            last = (ix != ixp1) | (lane == 15)
            plsc.addupdate_scatter(acc_v, [ix], p, mask=last)
            plsc.addupdate_scatter(acc_v, [ix], -e, mask=first)
            return 0
        lax.fori_loop(0, nvec, body, 0, unroll=4)

        pltpu.sync_copy(acc_v, shared.at[w])
        plsc.subcore_barrier()

        @pl.when(w < nred)
        def _reduce():
            pltpu.sync_copy(shared.at[:, pl.ds(w * cols, cols)], red_v)

            def rz(j, _):
                acc_v[pl.ds(j * 16, 16)] = zeros16
                return 0
            lax.fori_loop(0, cols // 16, rz, 0)

            def rbody(r, _):
                def rj(j, _2):
                    acc_v[pl.ds(j * 16, 16)] = (acc_v[pl.ds(j * 16, 16)] +
                                                red_v[r, pl.ds(j * 16, 16)])
                    return 0
                return lax.fori_loop(0, cols // 16, rj, 0)
            lax.fori_loop(0, NW, rbody, 0)

            pltpu.sync_copy(acc_v.at[pl.ds(0, cols)],
                            out_hbm.at[pl.ds(w * cols, cols)])

    return seg


def kernel(users, path, path_idx, all_embedding, edge_embedding,
           virtual_embedding, W_ih, W_hh, b_ih, b_hh, W_lin, b_lin):
    del users, virtual_embedding
    P = path.shape[0]
    all16 = jax.lax.slice(all_embedding, (0, 0), (_R16, _D))
    out = _tc_per_path(path, all16, edge_embedding, W_ih, W_hh, b_ih, b_hh,
                       W_lin, b_lin)
    score = _make_sc_segsum(P)(out, path_idx)
    return score.reshape(_B, 1)
